# Initial kernel scaffold; baseline (speedup 1.0000x reference)
import jax
import jax.numpy as jnp
from jax.experimental import pallas as pl


def kernel(keys, values, prev_key, prev_value):
    raise NotImplementedError("write your pallas kernel here")


# speedup vs baseline: 4.8056x; 4.8056x over previous
import functools

import jax
import jax.numpy as jnp
from jax import lax
from jax.experimental import pallas as pl
from jax.experimental.pallas import tpu as pltpu

_UPDATE_RATE = 0.1
_THRESH = 0.95
_EPS = 1e-12


def _argmax_body(nsteps, tile_n, bank_n,
                 keys_ref, prev_ref, idx_ref, corr_ref, bval_ref, bidx_ref):
    i = pl.program_id(0)

    @pl.when(i == 0)
    def _():
        bval_ref[...] = jnp.full_like(bval_ref[...], -3.0)
        bidx_ref[...] = jnp.zeros_like(bidx_ref[...])

    k = keys_ref[...]
    p = prev_ref[...]
    kn = jnp.sqrt(jnp.sum(k * k, axis=0, keepdims=True))
    knorm = (k / jnp.maximum(kn, _EPS)).astype(jnp.bfloat16)
    pn = jnp.sqrt(jnp.sum(p * p, axis=0, keepdims=True))
    pnorm = (p / jnp.maximum(pn, _EPS)).astype(jnp.bfloat16)
    corr = lax.dot_general(knorm, pnorm, (((0,), (0,)), ((), ())),
                           preferred_element_type=jnp.float32)
    rows = lax.broadcasted_iota(jnp.int32, corr.shape, 0)
    valid = (rows + i * tile_n) < bank_n
    corr = jnp.where(valid, corr, -2.0)
    tmax = jnp.max(corr, axis=0, keepdims=True)
    hit = corr == tmax
    targ = (jnp.min(jnp.where(hit, rows, tile_n), axis=0, keepdims=True)
            + i * tile_n)
    better = tmax > bval_ref[...]
    bval_ref[...] = jnp.where(better, tmax, bval_ref[...])
    bidx_ref[...] = jnp.where(better, targ, bidx_ref[...])

    @pl.when(i == nsteps - 1)
    def _():
        idx_ref[...] = bidx_ref[...]
        corr_ref[...] = bval_ref[...]


def _merge_body(tile_n, bank_n, d_key, n_prev,
                idx_ref, corr_ref, pk_ref, pv_ref, keys_ref, vals_ref,
                out_ref):
    i = pl.program_id(0)
    k = keys_ref[...]
    v = vals_ref[...]
    out_ref[:d_key, :] = k
    out_ref[d_key:, :] = v

    bc = corr_ref[...]

    @pl.when(jnp.max(bc) > _THRESH)
    def _():
        close = (bc > _THRESH).astype(jnp.float32)
        bi = idx_ref[...]
        colidx = (lax.broadcasted_iota(jnp.int32, (tile_n, n_prev), 0)
                  + i * tile_n)
        o_t = jnp.where(bi == colidx, close, 0.0)
        counts = lax.dot_general(jnp.ones((1, n_prev), jnp.float32), o_t,
                                 (((1,), (1,)), ((), ())),
                                 preferred_element_type=jnp.float32)
        pk = pk_ref[...]
        pv = pv_ref[...]
        pkn = jnp.sqrt(jnp.sum(pk * pk, axis=0, keepdims=True))
        npk = pk / jnp.maximum(pkn, _EPS)
        pvn = jnp.sqrt(jnp.sum(pv * pv, axis=0, keepdims=True))
        npv = pv / jnp.maximum(pvn, _EPS)
        sums_k = lax.dot_general(npk, o_t, (((1,), (1,)), ((), ())),
                                 preferred_element_type=jnp.float32)
        sums_v = lax.dot_general(npv, o_t, (((1,), (1,)), ((), ())),
                                 preferred_element_type=jnp.float32)
        safe = jnp.maximum(counts, 1.0)
        upd = counts > 0.0
        magk = jnp.sqrt(jnp.sum(k * k, axis=0, keepdims=True))
        normk = k / jnp.maximum(magk, _EPS)
        magv = jnp.sqrt(jnp.sum(v * v, axis=0, keepdims=True))
        normv = v / jnp.maximum(magv, _EPS)
        outk = jnp.where(upd, magk * ((1.0 - _UPDATE_RATE) * normk
                                      + _UPDATE_RATE * (sums_k / safe)), k)
        outv = jnp.where(upd, magv * ((1.0 - _UPDATE_RATE) * normv
                                      + _UPDATE_RATE * (sums_v / safe)), v)
        out_ref[:d_key, :] = outk
        out_ref[d_key:, :] = outv


def kernel(keys, values, prev_key, prev_value):
    d_key, bank_n = keys.shape
    d_val = values.shape[0]
    n_prev = prev_key.shape[1]
    tile_a = min(1024, bank_n)
    nsteps_a = pl.cdiv(bank_n, tile_a)

    best_idx, best_corr = pl.pallas_call(
        functools.partial(_argmax_body, nsteps_a, tile_a, bank_n),
        grid=(nsteps_a,),
        in_specs=[
            pl.BlockSpec((d_key, tile_a), lambda i: (0, i)),
            pl.BlockSpec((d_key, n_prev), lambda i: (0, 0)),
        ],
        out_specs=[
            pl.BlockSpec((1, n_prev), lambda i: (0, 0)),
            pl.BlockSpec((1, n_prev), lambda i: (0, 0)),
        ],
        out_shape=[
            jax.ShapeDtypeStruct((1, n_prev), jnp.int32),
            jax.ShapeDtypeStruct((1, n_prev), jnp.float32),
        ],
        scratch_shapes=[
            pltpu.VMEM((1, n_prev), jnp.float32),
            pltpu.VMEM((1, n_prev), jnp.int32),
        ],
    )(keys, prev_key)

    tile_c = min(1024, bank_n)
    nsteps_c = pl.cdiv(bank_n, tile_c)
    out = pl.pallas_call(
        functools.partial(_merge_body, tile_c, bank_n, d_key, n_prev),
        grid=(nsteps_c,),
        in_specs=[
            pl.BlockSpec((1, n_prev), lambda i: (0, 0)),
            pl.BlockSpec((1, n_prev), lambda i: (0, 0)),
            pl.BlockSpec((d_key, n_prev), lambda i: (0, 0)),
            pl.BlockSpec((d_val, n_prev), lambda i: (0, 0)),
            pl.BlockSpec((d_key, tile_c), lambda i: (0, i)),
            pl.BlockSpec((d_val, tile_c), lambda i: (0, i)),
        ],
        out_specs=pl.BlockSpec((d_key + d_val, tile_c), lambda i: (0, i)),
        out_shape=jax.ShapeDtypeStruct((d_key + d_val, bank_n), jnp.float32),
    )(best_idx, best_corr, prev_key, prev_value, keys, values)
    return out
